# baseline (speedup 1.0000x reference)
import jax
import jax.numpy as jnp
from jax.experimental import pallas as pl
from jax.experimental.pallas import tpu as pltpu

_CHUNK_H = 28
_NBUF = 4


def _body(t_hbm, o_hbm, inb, outb, insem, outsem):
    c, h = t_hbm.shape[0], t_hbm.shape[1]
    steps_per_c = h // _CHUNK_H
    steps = c * steps_per_c

    def src(i):
        return t_hbm.at[i // steps_per_c, pl.ds((i % steps_per_c) * _CHUNK_H, _CHUNK_H)]

    def dst(i):
        return o_hbm.at[i // steps_per_c, pl.ds((i % steps_per_c) * _CHUNK_H, _CHUNK_H)]

    def start_in(i):
        pltpu.make_async_copy(src(i), inb.at[i % _NBUF], insem.at[i % _NBUF]).start()

    def wait_in(i):
        pltpu.make_async_copy(src(i), inb.at[i % _NBUF], insem.at[i % _NBUF]).wait()

    def start_out(i):
        pltpu.make_async_copy(outb.at[i % _NBUF], dst(i), outsem.at[i % _NBUF]).start()

    def wait_out(i):
        pltpu.make_async_copy(outb.at[i % _NBUF], dst(i), outsem.at[i % _NBUF]).wait()

    for i in range(_NBUF):
        start_in(i)
    for i in range(steps):
        b = i % _NBUF
        wait_in(i)
        if i >= _NBUF:
            wait_out(i - _NBUF)
        x = jnp.clip(inb[b], 0.0, 1.0)
        outb[b] = jnp.round(x * 255.0) / 255.0
        start_out(i)
        if i + _NBUF < steps:
            start_in(i + _NBUF)
    for i in range(steps - _NBUF, steps):
        wait_out(i)


def kernel(watermark_samples, response):
    n, c, h, w = watermark_samples.shape
    t = jnp.transpose(watermark_samples, (1, 2, 3, 0))
    out = pl.pallas_call(
        _body,
        in_specs=[pl.BlockSpec(memory_space=pltpu.HBM)],
        out_specs=pl.BlockSpec(memory_space=pltpu.HBM),
        out_shape=jax.ShapeDtypeStruct((c, h, w, n), jnp.float32),
        scratch_shapes=[
            pltpu.VMEM((_NBUF, _CHUNK_H, w, n), jnp.float32),
            pltpu.VMEM((_NBUF, _CHUNK_H, w, n), jnp.float32),
            pltpu.SemaphoreType.DMA((_NBUF,)),
            pltpu.SemaphoreType.DMA((_NBUF,)),
        ],
    )(t)
    return (jnp.transpose(out, (3, 0, 1, 2)), response)
